# Initial kernel scaffold; baseline (speedup 1.0000x reference)
import jax
import jax.numpy as jnp
from jax.experimental import pallas as pl


def kernel(input_ids, embed_tokens_weight):
    raise NotImplementedError("write your pallas kernel here")


# speedup vs baseline: 1.5466x; 1.5466x over previous
import functools

import jax
import jax.numpy as jnp
from jax import lax
from jax.experimental import pallas as pl
from jax.experimental.pallas import tpu as pltpu
from jax.experimental.pallas import tpu_sc as plsc

HIDDEN = 1024
NUM_WORKERS = 32
CHUNK = 64


def _make_lookup(b_total: int):
    b_per_w = b_total // NUM_WORKERS
    n_chunk = b_per_w // CHUNK

    mesh = plsc.VectorSubcoreMesh(core_axis_name="c", subcore_axis_name="s")

    @functools.partial(
        pl.kernel,
        mesh=mesh,
        out_type=jax.ShapeDtypeStruct((b_total, HIDDEN), jnp.float32),
        scratch_types=[
            pltpu.VMEM((n_chunk, CHUNK), jnp.int32),
            pltpu.VMEM((CHUNK, HIDDEN), jnp.float32),
            pltpu.SemaphoreType.DMA,
        ],
    )
    def lookup(idx_hbm, table_hbm, out_hbm, idx_v, rows_v, sem):
        wid = lax.axis_index("s") * 2 + lax.axis_index("c")
        base = wid * b_per_w
        pltpu.sync_copy(idx_hbm.at[wid], idx_v)
        for c in range(n_chunk):
            pltpu.async_copy(table_hbm.at[idx_v.at[c]], rows_v, sem).wait()
            pltpu.sync_copy(rows_v, out_hbm.at[pl.ds(base + c * CHUNK, CHUNK)])

    return lookup


def kernel(input_ids, embed_tokens_weight):
    b, s = input_ids.shape
    b_total = b * s
    ids = input_ids.astype(jnp.int32).reshape(
        NUM_WORKERS, b_total // (NUM_WORKERS * CHUNK), CHUNK)
    out = _make_lookup(b_total)(ids, embed_tokens_weight)
    return out.reshape(b, s, HIDDEN)


# speedup vs baseline: 1.6150x; 1.0442x over previous
import functools

import jax
import jax.numpy as jnp
from jax import lax
from jax.experimental import pallas as pl
from jax.experimental.pallas import tpu as pltpu
from jax.experimental.pallas import tpu_sc as plsc

HIDDEN = 1024
NUM_WORKERS = 32
CHUNK = 32


def _make_lookup(b_total: int):
    b_per_w = b_total // NUM_WORKERS
    n_chunk = b_per_w // CHUNK

    mesh = plsc.VectorSubcoreMesh(core_axis_name="c", subcore_axis_name="s")

    @functools.partial(
        pl.kernel,
        mesh=mesh,
        out_type=jax.ShapeDtypeStruct((b_total, HIDDEN), jnp.float32),
        scratch_types=[
            pltpu.VMEM((n_chunk, CHUNK), jnp.int32),
            pltpu.VMEM((2, CHUNK, HIDDEN), jnp.float32),
            pltpu.SemaphoreType.DMA,
            pltpu.SemaphoreType.DMA,
            pltpu.SemaphoreType.DMA,
            pltpu.SemaphoreType.DMA,
        ],
    )
    def lookup(idx_hbm, table_hbm, out_hbm, idx_v, rows_v, g0, g1, s0, s1):
        wid = lax.axis_index("s") * 2 + lax.axis_index("c")
        base = wid * b_per_w
        gsem = (g0, g1)
        ssem = (s0, s1)
        pltpu.sync_copy(idx_hbm.at[wid], idx_v)

        gathers = [None] * n_chunk
        scatters = [None] * n_chunk
        gathers[0] = pltpu.async_copy(
            table_hbm.at[idx_v.at[0]], rows_v.at[0], gsem[0])
        for c in range(n_chunk):
            nxt = c + 1
            if nxt < n_chunk:
                if nxt >= 2:
                    scatters[nxt - 2].wait()
                gathers[nxt] = pltpu.async_copy(
                    table_hbm.at[idx_v.at[nxt]], rows_v.at[nxt % 2],
                    gsem[nxt % 2])
            gathers[c].wait()
            scatters[c] = pltpu.async_copy(
                rows_v.at[c % 2],
                out_hbm.at[pl.ds(base + c * CHUNK, CHUNK)],
                ssem[c % 2])
        scatters[n_chunk - 2].wait()
        scatters[n_chunk - 1].wait()

    return lookup


def kernel(input_ids, embed_tokens_weight):
    b, s = input_ids.shape
    b_total = b * s
    ids = input_ids.astype(jnp.int32).reshape(
        NUM_WORKERS, b_total // (NUM_WORKERS * CHUNK), CHUNK)
    out = _make_lookup(b_total)(ids, embed_tokens_weight)
    return out.reshape(b, s, HIDDEN)


# speedup vs baseline: 1.6482x; 1.0205x over previous
import functools

import jax
import jax.numpy as jnp
from jax import lax
from jax.experimental import pallas as pl
from jax.experimental.pallas import tpu as pltpu
from jax.experimental.pallas import tpu_sc as plsc

HIDDEN = 1024
NUM_WORKERS = 32
CHUNK = 32


def _make_lookup(b_total: int):
    b_per_w = b_total // NUM_WORKERS
    n_chunk = b_per_w // CHUNK

    mesh = plsc.VectorSubcoreMesh(core_axis_name="c", subcore_axis_name="s")

    @functools.partial(
        pl.kernel,
        mesh=mesh,
        out_type=jax.ShapeDtypeStruct((b_total, HIDDEN), jnp.float32),
        scratch_types=[
            pltpu.VMEM((n_chunk, CHUNK), jnp.int32),
            pltpu.VMEM((3, CHUNK, HIDDEN), jnp.float32),
            pltpu.SemaphoreType.DMA,
            pltpu.SemaphoreType.DMA,
            pltpu.SemaphoreType.DMA,
            pltpu.SemaphoreType.DMA,
            pltpu.SemaphoreType.DMA,
            pltpu.SemaphoreType.DMA,
        ],
    )
    def lookup(idx_hbm, table_hbm, out_hbm, idx_v, rows_v,
               g0, g1, g2, s0, s1, s2):
        wid = lax.axis_index("s") * 2 + lax.axis_index("c")
        base = wid * b_per_w
        gsem = (g0, g1, g2)
        ssem = (s0, s1, s2)
        pltpu.sync_copy(idx_hbm.at[wid], idx_v)

        gathers = [None] * n_chunk
        scatters = [None] * n_chunk
        gathers[0] = pltpu.async_copy(
            table_hbm.at[idx_v.at[0]], rows_v.at[0], gsem[0])
        gathers[1] = pltpu.async_copy(
            table_hbm.at[idx_v.at[1]], rows_v.at[1], gsem[1])
        for c in range(n_chunk):
            nxt = c + 2
            if nxt < n_chunk:
                if nxt >= 3:
                    scatters[nxt - 3].wait()
                gathers[nxt] = pltpu.async_copy(
                    table_hbm.at[idx_v.at[nxt]], rows_v.at[nxt % 3],
                    gsem[nxt % 3])
            gathers[c].wait()
            scatters[c] = pltpu.async_copy(
                rows_v.at[c % 3],
                out_hbm.at[pl.ds(base + c * CHUNK, CHUNK)],
                ssem[c % 3])
        scatters[n_chunk - 3].wait()
        scatters[n_chunk - 2].wait()
        scatters[n_chunk - 1].wait()

    return lookup


def kernel(input_ids, embed_tokens_weight):
    b, s = input_ids.shape
    b_total = b * s
    ids = input_ids.astype(jnp.int32).reshape(
        NUM_WORKERS, b_total // (NUM_WORKERS * CHUNK), CHUNK)
    out = _make_lookup(b_total)(ids, embed_tokens_weight)
    return out.reshape(b, s, HIDDEN)
